# baseline (speedup 1.0000x reference)
import functools

import jax
import jax.numpy as jnp
from jax import lax
from jax.experimental import pallas as pl
from jax.experimental.pallas import tpu as pltpu
from jax.experimental.pallas import tpu_sc as plsc

_B, _L, _D = 1024, 200, 128
_CH0 = 104
_CH1 = _L - _CH0
_NC, _NS = 2, 16
_NW = _NC * _NS
_BPW = _B // _NW
_SEC = 2 * _BPW
_NBUF = 4
_LANES = 16

_mesh = plsc.VectorSubcoreMesh(
    core_axis_name="c", subcore_axis_name="s", num_cores=_NC, num_subcores=_NS
)


def _span(s):
    return (0, _CH0) if s % 2 == 0 else (_CH0, _CH1)


@functools.partial(
    pl.kernel,
    out_type=jax.ShapeDtypeStruct((_B, _L, _D), jnp.float32),
    mesh=_mesh,
    scratch_types=[
        pltpu.VMEM((_SEC, _CH0), jnp.int32),
        pltpu.VMEM((_L, _D), jnp.float32),
        [pltpu.VMEM((_CH0, _D), jnp.float32) for _ in range(_NBUF)],
        [pltpu.SemaphoreType.DMA for _ in range(_NBUF)],
        [pltpu.SemaphoreType.DMA for _ in range(_NBUF)],
    ],
)
def _emb(x_hbm, tab_hbm, pos_hbm, out_hbm, idx_v, pos_v, bufs, gsem, wsem):
    wid = lax.axis_index("s") * _NC + lax.axis_index("c")
    bbase = wid * _BPW
    pltpu.sync_copy(x_hbm.at[pl.ds(wid * _SEC, _SEC)], idx_v)
    pltpu.sync_copy(pos_hbm, pos_v)

    def issue_gather(s):
        b = s % _NBUF
        return pltpu.async_copy(tab_hbm.at[idx_v.at[s]], bufs[b], gsem[b])

    def issue_write(s):
        b = s % _NBUF
        off, n = _span(s)
        dst = out_hbm.at[bbase + s // 2, pl.ds(off, n)]
        return pltpu.async_copy(bufs[b].at[pl.ds(0, n)], dst, wsem[b])

    def add_pos(s):
        b = s % _NBUF
        off, n = _span(s)

        def body(r, carry):
            for k in range(_D // _LANES):
                sl = pl.ds(k * _LANES, _LANES)
                plsc.addupdate(bufs[b].at[r, sl], pos_v[off + r, sl])
            return carry

        lax.fori_loop(0, n, body, 0, unroll=2)

    gathers = [None] * _SEC
    writes = [None] * _SEC
    gathers[0] = issue_gather(0)
    gathers[1] = issue_gather(1)
    for s in range(_SEC):
        gathers[s].wait()
        add_pos(s)
        writes[s] = issue_write(s)
        if s + 2 < _SEC:
            if s - 2 >= 0:
                writes[s - 2].wait()
            gathers[s + 2] = issue_gather(s + 2)
    writes[_SEC - 2].wait()
    writes[_SEC - 1].wait()


def kernel(x, token_table, pos_table):
    x_pad = jnp.pad(x, ((0, 0), (0, 2 * _CH0 - _L))).reshape(2 * _B, _CH0)
    return _emb(x_pad, token_table, pos_table)


# speedup vs baseline: 2.5288x; 2.5288x over previous
import functools

import jax
import jax.numpy as jnp
from jax import lax
from jax.experimental import pallas as pl
from jax.experimental.pallas import tpu as pltpu
from jax.experimental.pallas import tpu_sc as plsc

_B, _L, _D = 1024, 200, 128
_CH = 80
_ROWS = _B * _L
_NCHUNK = _ROWS // _CH
_NC, _NS = 2, 16
_NW = _NC * _NS
_SEC = _NCHUNK // _NW
_NBUF = 4
_LANES = 16

_mesh = plsc.VectorSubcoreMesh(
    core_axis_name="c", subcore_axis_name="s", num_cores=_NC, num_subcores=_NS
)


@functools.partial(
    pl.kernel,
    out_type=jax.ShapeDtypeStruct((_NCHUNK, _CH, _D), jnp.float32),
    mesh=_mesh,
    scratch_types=[
        pltpu.VMEM((_SEC, _CH), jnp.int32),
        pltpu.VMEM((_L, _D), jnp.float32),
        [pltpu.VMEM((_CH, _D), jnp.float32) for _ in range(_NBUF)],
        [pltpu.SemaphoreType.DMA for _ in range(_NBUF)],
        [pltpu.SemaphoreType.DMA for _ in range(_NBUF)],
    ],
)
def _emb(x_hbm, tab_hbm, pos_hbm, out_hbm, idx_v, pos_v, bufs, gsem, wsem):
    wid = lax.axis_index("s") * _NC + lax.axis_index("c")
    cbase = wid * _SEC
    pltpu.sync_copy(x_hbm.at[pl.ds(cbase, _SEC)], idx_v)
    pltpu.sync_copy(pos_hbm, pos_v)

    def issue_gather(s):
        b = s % _NBUF
        return pltpu.async_copy(tab_hbm.at[idx_v.at[s]], bufs[b], gsem[b])

    def issue_write(s):
        b = s % _NBUF
        return pltpu.async_copy(bufs[b], out_hbm.at[cbase + s], wsem[b])

    def add_pos(s):
        b = s % _NBUF
        off = lax.rem((cbase + s) * _CH, _L)

        def body(r, carry):
            p = lax.rem(off + r, _L)
            for k in range(_D // _LANES):
                sl = pl.ds(k * _LANES, _LANES)
                plsc.addupdate(bufs[b].at[r, sl], pos_v[p, sl])
            return carry

        lax.fori_loop(0, _CH, body, 0)

    gathers = [None] * _SEC
    writes = [None] * _SEC
    gathers[0] = issue_gather(0)
    gathers[1] = issue_gather(1)
    for s in range(_SEC):
        gathers[s].wait()
        add_pos(s)
        writes[s] = issue_write(s)
        if s + 2 < _SEC:
            if s - 2 >= 0:
                writes[s - 2].wait()
            gathers[s + 2] = issue_gather(s + 2)
    writes[_SEC - 2].wait()
    writes[_SEC - 1].wait()


def kernel(x, token_table, pos_table):
    x2 = x.reshape(_NCHUNK, _CH)
    out = _emb(x2, token_table, pos_table)
    return out.reshape(_B, _L, _D)
